# baseline (speedup 1.0000x reference)
import functools

import jax
import jax.numpy as jnp
from jax import lax
from jax.experimental import pallas as pl
from jax.experimental.pallas import tpu as pltpu
from jax.experimental.pallas import tpu_sc as plsc

TOP_K_PER_ROW = 64

NW = 32
NB1 = 4096
NB2 = 2048
NB3 = 256
SUBCAP = 2048
CAP = 16 * SUBCAP
SUBCAP2 = 256
CAP2 = 16 * SUBCAP2
WSZ = 16384
SAMPLE_WIN_STRIDE = 16
SAMPLE_VREG_STRIDE = 4
MARGIN = 3

_mesh = functools.partial(
    plsc.VectorSubcoreMesh, core_axis_name="c", subcore_axis_name="s",
    num_cores=2)

_sc_params = pltpu.CompilerParams(needs_layout_passes=False)


def _wid():
    return lax.axis_index("s") * 2 + lax.axis_index("c")


def _zero_i32(ref, n):
    def body(i, _):
        ref[pl.ds(i * 16, 16)] = jnp.zeros((16,), jnp.int32)
        return 0
    lax.fori_loop(0, n // 16, body, 0)


def _zero_f32(ref, n):
    def body(i, _):
        ref[pl.ds(i * 16, 16)] = jnp.zeros((16,), jnp.float32)
        return 0
    lax.fori_loop(0, n // 16, body, 0)


def _extract(vec, lane):
    lanes = lax.iota(jnp.int32, 16)
    return jnp.sum(jnp.where(lanes == lane, vec, 0))


def _rank_search(href, nb, rank):
    nv = nb // 16
    lanes = lax.iota(jnp.int32, 16)

    def body(j, carry):
        found_bin, found_gt, cum_above = carry
        i = nv - 1 - j
        v = href[pl.ds(i * 16, 16)]
        incl = jnp.cumsum(v)
        s = jnp.sum(v)
        gt = cum_above + (s - incl)
        m = (gt < rank) & (gt + v >= rank)
        mi = jnp.where(m, 1, 0)
        hit = jnp.sum(mi)
        lane = jnp.sum(jnp.where(m, lanes, 0))
        gt_at = jnp.sum(jnp.where(m, gt, 0))
        found_bin = jnp.where(hit > 0, i * 16 + lane, found_bin)
        found_gt = jnp.where(hit > 0, gt_at, found_gt)
        return found_bin, found_gt, cum_above + s

    out = lax.fori_loop(0, nv, body,
                        (jnp.int32(0), jnp.int32(0), jnp.int32(0)))
    return out[0], out[1]


def _sc_compact(acts, k):
    B, D = acts.shape
    n = B * D
    rows = B // NW
    WR, WC = 8, 2048
    nwg_r = rows // WR
    nwg_c = D // WC
    nwin = nwg_r * nwg_c
    assert nwin % 2 == 0 or nwin == 1
    wsz = WR * WC
    samp_wins = list(range(0, nwin, SAMPLE_WIN_STRIDE))
    samp_elems = len(samp_wins) * (wsz // (16 * SAMPLE_VREG_STRIDE)) * 16
    r_samp = max(1, -(-(MARGIN * k * samp_elems) // n))

    @functools.partial(
        pl.kernel, mesh=_mesh(),
        compiler_params=pltpu.CompilerParams(needs_layout_passes=False,
                                             use_tc_tiling_on_sc=True),
        out_type=(jax.ShapeDtypeStruct((NW, CAP), jnp.float32),
                  jax.ShapeDtypeStruct((NW, NB1), jnp.int32)),
        scratch_types=[pltpu.VMEM((WR, WC), jnp.float32),
                       pltpu.VMEM((WR, WC), jnp.float32),
                       pltpu.VMEM((CAP,), jnp.float32),
                       pltpu.VMEM((NB1,), jnp.int32),
                       pltpu.SemaphoreType.DMA,
                       pltpu.SemaphoreType.DMA],
    )
    def compact_kernel(acts_hbm, cands_hbm, hist_hbm, win0, win1, vals,
                       hist, sem0, sem1):
        w = _wid()
        row_base = w * rows
        ones = jnp.ones((16,), jnp.int32)
        lanes = lax.iota(jnp.int32, 16)

        def src(g):
            gr = g // nwg_c
            gc = g % nwg_c
            return acts_hbm.at[pl.ds(row_base + gr * WR, WR),
                               pl.ds(gc * WC, WC)]

        _zero_i32(hist, NB1)

        def samp_win(gs, _):
            pltpu.sync_copy(src(gs * SAMPLE_WIN_STRIDE), win0)

            def samp_row(r, _):
                def samp_body(i, _):
                    for u in range(4):
                        c = (i * 4 + u) * SAMPLE_VREG_STRIDE * 16
                        v = win0[r, pl.ds(c, 16)]
                        bits = lax.bitcast_convert_type(v, jnp.int32)
                        b = jnp.right_shift(bits, 19)
                        plsc.addupdate_scatter(hist, [b], ones,
                                               mask=v > 0.0)
                    return 0
                lax.fori_loop(0, WC // (16 * SAMPLE_VREG_STRIDE * 4),
                              samp_body, 0)
                return 0
            lax.fori_loop(0, WR, samp_row, 0)
            return 0
        lax.fori_loop(0, len(samp_wins), samp_win, 0)
        bf, _gt = _rank_search(hist, NB1, jnp.int32(r_samp))
        floor_f = lax.bitcast_convert_type(
            jnp.full((16,), bf * 524288, jnp.int32), jnp.float32)

        _zero_f32(vals, CAP)
        lane_base = lanes * SUBCAP

        def start(g, buf, sem):
            pltpu.async_copy(src(g), buf, sem)

        def wait(g, buf, sem):
            pltpu.make_async_copy(src(g), buf, sem).wait()

        def process(buf, cnt_vec):
            unroll = 8

            def row_step(r, cnt):
                def vec_body(i, cnt):
                    vs, ms, mis = [], [], []
                    for u in range(unroll):
                        v = buf[r, pl.ds(i * (16 * unroll) + u * 16, 16)]
                        m = v >= floor_f
                        vs.append(v)
                        ms.append(m)
                        mis.append(jnp.where(m, 1, 0))
                    base = lane_base + jnp.minimum(cnt,
                                                   SUBCAP - 1 - unroll)
                    pre = base
                    for u in range(unroll):
                        plsc.store_scatter(vals, [pre], vs[u], mask=ms[u])
                        pre = pre + mis[u]
                    return cnt + (pre - base)
                return lax.fori_loop(0, WC // (16 * unroll), vec_body, cnt)
            return lax.fori_loop(0, WR, row_step, cnt_vec)

        start(0, win0, sem0)
        if nwin > 1:
            start(1, win1, sem1)

        def pair_body(h, cnt_vec):
            g0 = h * 2
            wait(g0, win0, sem0)
            cnt_vec = process(win0, cnt_vec)

            @pl.when(g0 + 2 < nwin)
            def _():
                start(g0 + 2, win0, sem0)
            wait(g0 + 1, win1, sem1)
            cnt_vec = process(win1, cnt_vec)

            @pl.when(g0 + 3 < nwin)
            def _():
                start(g0 + 3, win1, sem1)
            return cnt_vec

        cnt0 = jnp.zeros((16,), jnp.int32)
        if nwin == 1:
            wait(0, win0, sem0)
            process(win0, cnt0)
        else:
            lax.fori_loop(0, nwin // 2, pair_body, cnt0)

        _zero_i32(hist, NB1)

        def chist_body(i, _):
            for u in range(4):
                v = vals[pl.ds((i * 4 + u) * 16, 16)]
                bits = lax.bitcast_convert_type(v, jnp.int32)
                b = jnp.right_shift(bits, 19)
                plsc.addupdate_scatter(hist, [b], ones, mask=v >= floor_f)
            return 0
        lax.fori_loop(0, CAP // 64, chist_body, 0)

        pltpu.sync_copy(vals, cands_hbm.at[w])
        pltpu.sync_copy(hist, hist_hbm.at[w])

    return compact_kernel(acts)


def _rows_pipelined(hbm, buf0, buf1, sem0, sem1, nrows, process):
    def start(r, buf, sem):
        pltpu.async_copy(hbm.at[r], buf, sem)

    def wait(r, buf, sem):
        pltpu.make_async_copy(hbm.at[r], buf, sem).wait()

    start(0, buf0, sem0)
    if nrows > 1:
        start(1, buf1, sem1)

    def pair(h, _):
        r0 = h * 2
        wait(r0, buf0, sem0)
        process(buf0)

        @pl.when(r0 + 2 < nrows)
        def _():
            start(r0 + 2, buf0, sem0)
        wait(r0 + 1, buf1, sem1)
        process(buf1)

        @pl.when(r0 + 3 < nrows)
        def _():
            start(r0 + 3, buf1, sem1)
        return 0
    lax.fori_loop(0, nrows // 2, pair, 0)


def _add_rows(acc, row, nb):
    def body(i, _):
        for u in range(4):
            sl = pl.ds((i * 4 + u) * 16, 16)
            acc[sl] = acc[sl] + row[sl]
        return 0
    lax.fori_loop(0, nb // 64, body, 0)


def _sc_bracket(hists, k):
    @functools.partial(
        pl.kernel, mesh=_mesh(), compiler_params=_sc_params,
        out_type=jax.ShapeDtypeStruct((16,), jnp.int32),
        scratch_types=[pltpu.VMEM((NB1,), jnp.int32),
                       pltpu.VMEM((NW // 2, NB1), jnp.int32),
                       pltpu.VMEM((16,), jnp.int32),
                       pltpu.SemaphoreType.DMA,
                       pltpu.SemaphoreType.DMA],
    )
    def bracket_kernel(hists_hbm, out_hbm, acc, blk, res, sem0, sem1):
        w = _wid()

        @pl.when(w == 0)
        def _():
            _zero_i32(acc, NB1)
            for h in range(2):
                pltpu.sync_copy(hists_hbm.at[pl.ds(h * (NW // 2),
                                                   NW // 2)], blk)

                def row_body(r, _):
                    def add_body(i, _):
                        for u in range(4):
                            sl = pl.ds((i * 4 + u) * 16, 16)
                            acc[sl] = acc[sl] + blk[r, sl]
                        return 0
                    lax.fori_loop(0, NB1 // 64, add_body, 0)
                    return 0
                lax.fori_loop(0, NW // 2, row_body, 0)

            b1, gt1 = _rank_search(acc, NB1, jnp.int32(k))
            lanes = lax.iota(jnp.int32, 16)
            res[...] = jnp.where(lanes == 0, b1,
                                 jnp.where(lanes == 1, gt1, 0))
            pltpu.sync_copy(res, out_hbm)

    return bracket_kernel(hists)


def _sc_filter(cands, bracket):
    @functools.partial(
        pl.kernel, mesh=_mesh(), compiler_params=_sc_params,
        out_type=(jax.ShapeDtypeStruct((NW, CAP2), jnp.float32),
                  jax.ShapeDtypeStruct((NW, NB2), jnp.int32)),
        scratch_types=[pltpu.VMEM((CAP,), jnp.float32),
                       pltpu.VMEM((CAP2,), jnp.float32),
                       pltpu.VMEM((NB2,), jnp.int32),
                       pltpu.VMEM((16,), jnp.int32)],
    )
    def filter_kernel(cands_hbm, brk_hbm, inbin_hbm, h2_hbm, cbuf, inbin,
                      h2, binfo):
        w = _wid()
        pltpu.sync_copy(cands_hbm.at[w], cbuf)
        pltpu.sync_copy(brk_hbm, binfo)
        b1 = _extract(binfo[...], 0)
        b1v = jnp.full((16,), b1, jnp.int32)
        lanes = lax.iota(jnp.int32, 16)
        lane_base = lanes * SUBCAP2
        ones = jnp.ones((16,), jnp.int32)

        _zero_f32(inbin, CAP2)

        def body(i, cnt):
            for u in range(4):
                v = cbuf[pl.ds((i * 4 + u) * 16, 16)]
                bits = lax.bitcast_convert_type(v, jnp.int32)
                m = jnp.right_shift(bits, 19) == b1v
                mi = jnp.where(m, 1, 0)
                idx = lane_base + jnp.minimum(cnt, SUBCAP2 - 1)
                plsc.store_scatter(inbin, [idx], v, mask=m)
                cnt = cnt + mi
            return cnt
        lax.fori_loop(0, CAP // 64, body, jnp.zeros((16,), jnp.int32))

        _zero_i32(h2, NB2)

        def h2_body(i, _):
            v = inbin[pl.ds(i * 16, 16)]
            bits = lax.bitcast_convert_type(v, jnp.int32)
            m = jnp.right_shift(bits, 19) == b1v
            b2 = jnp.right_shift(bits, 8) & 0x7FF
            plsc.addupdate_scatter(h2, [b2], ones, mask=m)
            return 0
        lax.fori_loop(0, CAP2 // 16, h2_body, 0)

        pltpu.sync_copy(inbin, inbin_hbm.at[w])
        pltpu.sync_copy(h2, h2_hbm.at[w])

    return filter_kernel(cands, bracket)


def _sc_final(inbin, h2s, bracket, k):
    @functools.partial(
        pl.kernel, mesh=_mesh(), compiler_params=_sc_params,
        out_type=jax.ShapeDtypeStruct((16,), jnp.float32),
        scratch_types=[pltpu.VMEM((NB2,), jnp.int32),
                       pltpu.VMEM((NW, NB2), jnp.int32),
                       pltpu.VMEM((NW // 4, CAP2), jnp.float32),
                       pltpu.VMEM((NB3,), jnp.int32),
                       pltpu.VMEM((16,), jnp.int32),
                       pltpu.VMEM((16,), jnp.float32)],
    )
    def final_kernel(inbin_hbm, h2s_hbm, brk_hbm, out_hbm, acc, h2all,
                     cbuf, h3, binfo, res):
        w = _wid()

        @pl.when(w == 0)
        def _():
            pltpu.sync_copy(brk_hbm, binfo)
            b1 = _extract(binfo[...], 0)
            gt1 = _extract(binfo[...], 1)
            rank1 = jnp.int32(k) - gt1
            b1v = jnp.full((16,), b1, jnp.int32)
            ones = jnp.ones((16,), jnp.int32)

            _zero_i32(acc, NB2)
            pltpu.sync_copy(h2s_hbm, h2all)

            def row_body(r, _):
                def add_body(i, _):
                    for u in range(4):
                        sl = pl.ds((i * 4 + u) * 16, 16)
                        acc[sl] = acc[sl] + h2all[r, sl]
                    return 0
                lax.fori_loop(0, NB2 // 64, add_body, 0)
                return 0
            lax.fori_loop(0, NW, row_body, 0)

            b2, gt2 = _rank_search(acc, NB2, rank1)
            rank2 = rank1 - gt2
            b2v = jnp.full((16,), b2, jnp.int32)

            _zero_i32(h3, NB3)
            for q in range(4):
                pltpu.sync_copy(inbin_hbm.at[pl.ds(q * (NW // 4),
                                                   NW // 4)], cbuf)

                def h3_row(r, _):
                    def h3_body(i, _):
                        for u in range(4):
                            v = cbuf[r, pl.ds((i * 4 + u) * 16, 16)]
                            bits = lax.bitcast_convert_type(v, jnp.int32)
                            m = ((jnp.right_shift(bits, 19) == b1v)
                                 & ((jnp.right_shift(bits, 8) & 0x7FF)
                                    == b2v))
                            b3 = bits & 0xFF
                            plsc.addupdate_scatter(h3, [b3], ones, mask=m)
                        return 0
                    lax.fori_loop(0, CAP2 // 64, h3_body, 0)
                    return 0
                lax.fori_loop(0, NW // 4, h3_row, 0)

            b3, _gt3 = _rank_search(h3, NB3, rank2)
            tbits = (b1 * 524288) + (b2 * 256) + b3
            res[...] = lax.bitcast_convert_type(
                jnp.full((16,), tbits, jnp.int32), jnp.float32)
            pltpu.sync_copy(res, out_hbm)

    return final_kernel(inbin, h2s, bracket)


def _select_threshold(acts, k):
    cands, hists = _sc_compact(acts, k)
    bracket = _sc_bracket(hists, k)
    inbin, h2s = _sc_filter(cands, bracket)
    tvec = _sc_final(inbin, h2s, bracket, k)
    return tvec[0:1]


def _encode_body(x_ref, w_ref, b_ref, out_ref):
    acc = jnp.dot(x_ref[...], w_ref[...],
                  preferred_element_type=jnp.float32,
                  precision=jax.lax.Precision.DEFAULT)
    out_ref[...] = jnp.maximum(acc + b_ref[...], 0.0)


def _encode(x, w_enc, b_enc, bn=1024):
    B, d_src = x.shape
    D = w_enc.shape[1]
    grid = (D // bn,)
    return pl.pallas_call(
        _encode_body,
        grid=grid,
        in_specs=[
            pl.BlockSpec((B, d_src), lambda j: (0, 0)),
            pl.BlockSpec((d_src, bn), lambda j: (0, j)),
            pl.BlockSpec((1, bn), lambda j: (0, j)),
        ],
        out_specs=pl.BlockSpec((B, bn), lambda j: (0, j)),
        out_shape=jax.ShapeDtypeStruct((B, D), jnp.float32),
    )(x, w_enc, b_enc.reshape(1, D))


def _decode_body(t_ref, a_ref, w_ref, bd_ref, out_ref):
    kstep = pl.program_id(1)
    t = t_ref[0]
    a = a_ref[...]
    a = jnp.where(a >= t, a, 0.0)
    contrib = jnp.dot(a, w_ref[...],
                      preferred_element_type=jnp.float32,
                      precision=jax.lax.Precision.DEFAULT)

    @pl.when(kstep == 0)
    def _():
        out_ref[...] = contrib + bd_ref[...]

    @pl.when(kstep > 0)
    def _():
        out_ref[...] += contrib


def _decode(threshold, acts, w_dec, b_dec, bm=2048, bk=512):
    B, D = acts.shape
    d_tgt = w_dec.shape[1]
    bm = min(bm, B)
    grid = (B // bm, D // bk)
    return pl.pallas_call(
        _decode_body,
        grid=grid,
        in_specs=[
            pl.BlockSpec(memory_space=pltpu.SMEM),
            pl.BlockSpec((bm, bk), lambda i, j: (i, j)),
            pl.BlockSpec((bk, d_tgt), lambda i, j: (j, 0)),
            pl.BlockSpec((1, d_tgt), lambda i, j: (0, 0)),
        ],
        out_specs=pl.BlockSpec((bm, d_tgt), lambda i, j: (i, 0)),
        out_shape=jax.ShapeDtypeStruct((B, d_tgt), jnp.float32),
    )(threshold, acts, w_dec, b_dec.reshape(1, d_tgt))


def kernel(x_source, x_target, W_enc, b_enc, W_dec, b_dec):
    B = x_source.shape[0]
    D = W_enc.shape[1]
    bn = min(1024, D)
    acts = _encode(x_source, W_enc, b_enc, bn=bn)
    k = TOP_K_PER_ROW * B
    threshold = _select_threshold(acts, k)
    return _decode(threshold, acts, W_dec, b_dec, bk=min(512, D))


# speedup vs baseline: 1.0279x; 1.0279x over previous
import functools

import jax
import jax.numpy as jnp
from jax import lax
from jax.experimental import pallas as pl
from jax.experimental.pallas import tpu as pltpu
from jax.experimental.pallas import tpu_sc as plsc

TOP_K_PER_ROW = 64

NW = 32
NB1 = 4096
NB2 = 2048
NB3 = 256
SUBCAP = 2048
CAP = 16 * SUBCAP
SUBCAP2 = 256
CAP2 = 16 * SUBCAP2
WSZ = 16384
SAMPLE_WIN_STRIDE = 8
SAMPLE_VREG_STRIDE = 4
MARGIN = 3

_mesh = functools.partial(
    plsc.VectorSubcoreMesh, core_axis_name="c", subcore_axis_name="s",
    num_cores=2)

_sc_params = pltpu.CompilerParams(needs_layout_passes=False)


def _wid():
    return lax.axis_index("s") * 2 + lax.axis_index("c")


def _zero_i32(ref, n):
    def body(i, _):
        ref[pl.ds(i * 16, 16)] = jnp.zeros((16,), jnp.int32)
        return 0
    lax.fori_loop(0, n // 16, body, 0)


def _zero_f32(ref, n):
    def body(i, _):
        ref[pl.ds(i * 16, 16)] = jnp.zeros((16,), jnp.float32)
        return 0
    lax.fori_loop(0, n // 16, body, 0)


def _extract(vec, lane):
    lanes = lax.iota(jnp.int32, 16)
    return jnp.sum(jnp.where(lanes == lane, vec, 0))


def _rank_search(href, nb, rank):
    nv = nb // 16
    lanes = lax.iota(jnp.int32, 16)

    def body(j, carry):
        found_bin, found_gt, cum_above = carry
        i = nv - 1 - j
        v = href[pl.ds(i * 16, 16)]
        incl = jnp.cumsum(v)
        s = jnp.sum(v)
        gt = cum_above + (s - incl)
        m = (gt < rank) & (gt + v >= rank)
        mi = jnp.where(m, 1, 0)
        hit = jnp.sum(mi)
        lane = jnp.sum(jnp.where(m, lanes, 0))
        gt_at = jnp.sum(jnp.where(m, gt, 0))
        found_bin = jnp.where(hit > 0, i * 16 + lane, found_bin)
        found_gt = jnp.where(hit > 0, gt_at, found_gt)
        return found_bin, found_gt, cum_above + s

    out = lax.fori_loop(0, nv, body,
                        (jnp.int32(0), jnp.int32(0), jnp.int32(0)))
    return out[0], out[1]


def _sc_compact(acts, k):
    B, D = acts.shape
    n = B * D
    rows = B // NW
    WR, WC = 8, 2048
    nwg_r = rows // WR
    nwg_c = D // WC
    nwin = nwg_r * nwg_c
    assert nwin % 2 == 0 or nwin == 1
    wsz = WR * WC
    samp_wins = list(range(0, nwin, SAMPLE_WIN_STRIDE))
    samp_elems = len(samp_wins) * (wsz // (16 * SAMPLE_VREG_STRIDE)) * 16
    r_samp = max(1, -(-(MARGIN * k * samp_elems) // n))

    @functools.partial(
        pl.kernel, mesh=_mesh(),
        compiler_params=pltpu.CompilerParams(needs_layout_passes=False,
                                             use_tc_tiling_on_sc=True),
        out_type=(jax.ShapeDtypeStruct((NW, CAP), jnp.float32),
                  jax.ShapeDtypeStruct((NW, NB1), jnp.int32)),
        scratch_types=[pltpu.VMEM((WR, WC), jnp.float32),
                       pltpu.VMEM((WR, WC), jnp.float32),
                       pltpu.VMEM((CAP,), jnp.float32),
                       pltpu.VMEM((NB1,), jnp.int32),
                       pltpu.SemaphoreType.DMA,
                       pltpu.SemaphoreType.DMA],
    )
    def compact_kernel(acts_hbm, cands_hbm, hist_hbm, win0, win1, vals,
                       hist, sem0, sem1):
        w = _wid()
        row_base = w * rows
        ones = jnp.ones((16,), jnp.int32)
        lanes = lax.iota(jnp.int32, 16)

        def src(g):
            gr = g // nwg_c
            gc = g % nwg_c
            return acts_hbm.at[pl.ds(row_base + gr * WR, WR),
                               pl.ds(gc * WC, WC)]

        _zero_i32(hist, NB1)

        def samp_win(gs, _):
            pltpu.sync_copy(src(gs * SAMPLE_WIN_STRIDE), win0)

            def samp_row(r, _):
                def samp_body(i, _):
                    for u in range(4):
                        c = (i * 4 + u) * SAMPLE_VREG_STRIDE * 16
                        v = win0[r, pl.ds(c, 16)]
                        bits = lax.bitcast_convert_type(v, jnp.int32)
                        b = jnp.right_shift(bits, 19)
                        plsc.addupdate_scatter(hist, [b], ones,
                                               mask=v > 0.0)
                    return 0
                lax.fori_loop(0, WC // (16 * SAMPLE_VREG_STRIDE * 4),
                              samp_body, 0)
                return 0
            lax.fori_loop(0, WR, samp_row, 0)
            return 0
        lax.fori_loop(0, len(samp_wins), samp_win, 0)
        bf, _gt = _rank_search(hist, NB1, jnp.int32(r_samp))
        floor_f = lax.bitcast_convert_type(
            jnp.full((16,), bf * 524288, jnp.int32), jnp.float32)

        _zero_f32(vals, CAP)
        lane_base = lanes * SUBCAP

        def start(g, buf, sem):
            pltpu.async_copy(src(g), buf, sem)

        def wait(g, buf, sem):
            pltpu.make_async_copy(src(g), buf, sem).wait()

        def process(buf, cnt_vec):
            unroll = 8

            def row_step(r, cnt):
                def vec_body(i, cnt):
                    vs, ms, mis = [], [], []
                    for u in range(unroll):
                        v = buf[r, pl.ds(i * (16 * unroll) + u * 16, 16)]
                        m = v >= floor_f
                        vs.append(v)
                        ms.append(m)
                        mis.append(jnp.where(m, 1, 0))
                    base = lane_base + jnp.minimum(cnt,
                                                   SUBCAP - 1 - unroll)
                    pre = base
                    for u in range(unroll):
                        plsc.store_scatter(vals, [pre], vs[u], mask=ms[u])
                        pre = pre + mis[u]
                    return cnt + (pre - base)
                return lax.fori_loop(0, WC // (16 * unroll), vec_body, cnt)
            return lax.fori_loop(0, WR, row_step, cnt_vec)

        start(0, win0, sem0)
        if nwin > 1:
            start(1, win1, sem1)

        def pair_body(h, cnt_vec):
            g0 = h * 2
            wait(g0, win0, sem0)
            cnt_vec = process(win0, cnt_vec)

            @pl.when(g0 + 2 < nwin)
            def _():
                start(g0 + 2, win0, sem0)
            wait(g0 + 1, win1, sem1)
            cnt_vec = process(win1, cnt_vec)

            @pl.when(g0 + 3 < nwin)
            def _():
                start(g0 + 3, win1, sem1)
            return cnt_vec

        cnt0 = jnp.zeros((16,), jnp.int32)
        if nwin == 1:
            wait(0, win0, sem0)
            process(win0, cnt0)
        else:
            lax.fori_loop(0, nwin // 2, pair_body, cnt0)

        _zero_i32(hist, NB1)

        def chist_body(i, _):
            for u in range(4):
                v = vals[pl.ds((i * 4 + u) * 16, 16)]
                bits = lax.bitcast_convert_type(v, jnp.int32)
                b = jnp.right_shift(bits, 19)
                plsc.addupdate_scatter(hist, [b], ones, mask=v >= floor_f)
            return 0
        lax.fori_loop(0, CAP // 64, chist_body, 0)

        pltpu.sync_copy(vals, cands_hbm.at[w])
        pltpu.sync_copy(hist, hist_hbm.at[w])

    return compact_kernel(acts)


def _rows_pipelined(hbm, buf0, buf1, sem0, sem1, nrows, process):
    def start(r, buf, sem):
        pltpu.async_copy(hbm.at[r], buf, sem)

    def wait(r, buf, sem):
        pltpu.make_async_copy(hbm.at[r], buf, sem).wait()

    start(0, buf0, sem0)
    if nrows > 1:
        start(1, buf1, sem1)

    def pair(h, _):
        r0 = h * 2
        wait(r0, buf0, sem0)
        process(buf0)

        @pl.when(r0 + 2 < nrows)
        def _():
            start(r0 + 2, buf0, sem0)
        wait(r0 + 1, buf1, sem1)
        process(buf1)

        @pl.when(r0 + 3 < nrows)
        def _():
            start(r0 + 3, buf1, sem1)
        return 0
    lax.fori_loop(0, nrows // 2, pair, 0)


def _add_rows(acc, row, nb):
    def body(i, _):
        for u in range(4):
            sl = pl.ds((i * 4 + u) * 16, 16)
            acc[sl] = acc[sl] + row[sl]
        return 0
    lax.fori_loop(0, nb // 64, body, 0)


def _sc_bracket(hists, k):
    @functools.partial(
        pl.kernel, mesh=_mesh(), compiler_params=_sc_params,
        out_type=jax.ShapeDtypeStruct((16,), jnp.int32),
        scratch_types=[pltpu.VMEM((NB1,), jnp.int32),
                       pltpu.VMEM((NB1,), jnp.int32),
                       pltpu.VMEM((NB1,), jnp.int32),
                       pltpu.VMEM((16,), jnp.int32),
                       pltpu.SemaphoreType.DMA,
                       pltpu.SemaphoreType.DMA],
    )
    def bracket_kernel(hists_hbm, out_hbm, acc, row0, row1, res, sem0,
                       sem1):
        w = _wid()

        @pl.when(w == 0)
        def _():
            _zero_i32(acc, NB1)
            _rows_pipelined(hists_hbm, row0, row1, sem0, sem1, NW,
                            lambda row: _add_rows(acc, row, NB1))

            b1, gt1 = _rank_search(acc, NB1, jnp.int32(k))
            lanes = lax.iota(jnp.int32, 16)
            res[...] = jnp.where(lanes == 0, b1,
                                 jnp.where(lanes == 1, gt1, 0))
            pltpu.sync_copy(res, out_hbm)

    return bracket_kernel(hists)


def _sc_filter(cands, bracket):
    @functools.partial(
        pl.kernel, mesh=_mesh(), compiler_params=_sc_params,
        out_type=(jax.ShapeDtypeStruct((NW, CAP2), jnp.float32),
                  jax.ShapeDtypeStruct((NW, NB2), jnp.int32)),
        scratch_types=[pltpu.VMEM((CAP,), jnp.float32),
                       pltpu.VMEM((CAP2,), jnp.float32),
                       pltpu.VMEM((NB2,), jnp.int32),
                       pltpu.VMEM((16,), jnp.int32)],
    )
    def filter_kernel(cands_hbm, brk_hbm, inbin_hbm, h2_hbm, cbuf, inbin,
                      h2, binfo):
        w = _wid()
        pltpu.sync_copy(cands_hbm.at[w], cbuf)
        pltpu.sync_copy(brk_hbm, binfo)
        b1 = _extract(binfo[...], 0)
        b1v = jnp.full((16,), b1, jnp.int32)
        lanes = lax.iota(jnp.int32, 16)
        lane_base = lanes * SUBCAP2
        ones = jnp.ones((16,), jnp.int32)

        _zero_f32(inbin, CAP2)

        def body(i, cnt):
            for u in range(4):
                v = cbuf[pl.ds((i * 4 + u) * 16, 16)]
                bits = lax.bitcast_convert_type(v, jnp.int32)
                m = jnp.right_shift(bits, 19) == b1v
                mi = jnp.where(m, 1, 0)
                idx = lane_base + jnp.minimum(cnt, SUBCAP2 - 1)
                plsc.store_scatter(inbin, [idx], v, mask=m)
                cnt = cnt + mi
            return cnt
        lax.fori_loop(0, CAP // 64, body, jnp.zeros((16,), jnp.int32))

        _zero_i32(h2, NB2)

        def h2_body(i, _):
            v = inbin[pl.ds(i * 16, 16)]
            bits = lax.bitcast_convert_type(v, jnp.int32)
            m = jnp.right_shift(bits, 19) == b1v
            b2 = jnp.right_shift(bits, 8) & 0x7FF
            plsc.addupdate_scatter(h2, [b2], ones, mask=m)
            return 0
        lax.fori_loop(0, CAP2 // 16, h2_body, 0)

        pltpu.sync_copy(inbin, inbin_hbm.at[w])
        pltpu.sync_copy(h2, h2_hbm.at[w])

    return filter_kernel(cands, bracket)


def _sc_final(inbin, h2s, bracket, k):
    @functools.partial(
        pl.kernel, mesh=_mesh(), compiler_params=_sc_params,
        out_type=jax.ShapeDtypeStruct((16,), jnp.float32),
        scratch_types=[pltpu.VMEM((NB2,), jnp.int32),
                       pltpu.VMEM((NB2,), jnp.int32),
                       pltpu.VMEM((NB2,), jnp.int32),
                       pltpu.VMEM((CAP2,), jnp.float32),
                       pltpu.VMEM((CAP2,), jnp.float32),
                       pltpu.VMEM((NB3,), jnp.int32),
                       pltpu.VMEM((16,), jnp.int32),
                       pltpu.VMEM((16,), jnp.float32),
                       pltpu.SemaphoreType.DMA,
                       pltpu.SemaphoreType.DMA],
    )
    def final_kernel(inbin_hbm, h2s_hbm, brk_hbm, out_hbm, acc, row0, row1,
                     cbuf0, cbuf1, h3, binfo, res, sem0, sem1):
        w = _wid()

        @pl.when(w == 0)
        def _():
            pltpu.sync_copy(brk_hbm, binfo)
            b1 = _extract(binfo[...], 0)
            gt1 = _extract(binfo[...], 1)
            rank1 = jnp.int32(k) - gt1
            b1v = jnp.full((16,), b1, jnp.int32)
            ones = jnp.ones((16,), jnp.int32)

            _zero_i32(acc, NB2)
            _rows_pipelined(h2s_hbm, row0, row1, sem0, sem1, NW,
                            lambda row: _add_rows(acc, row, NB2))

            b2, gt2 = _rank_search(acc, NB2, rank1)
            rank2 = rank1 - gt2
            b2v = jnp.full((16,), b2, jnp.int32)

            _zero_i32(h3, NB3)

            def h3_rows(cbuf):
                def h3_body(i, _):
                    for u in range(4):
                        v = cbuf[pl.ds((i * 4 + u) * 16, 16)]
                        bits = lax.bitcast_convert_type(v, jnp.int32)
                        m = ((jnp.right_shift(bits, 19) == b1v)
                             & ((jnp.right_shift(bits, 8) & 0x7FF) == b2v))
                        b3 = bits & 0xFF
                        plsc.addupdate_scatter(h3, [b3], ones, mask=m)
                    return 0
                lax.fori_loop(0, CAP2 // 64, h3_body, 0)
            _rows_pipelined(inbin_hbm, cbuf0, cbuf1, sem0, sem1, NW,
                            h3_rows)

            b3, _gt3 = _rank_search(h3, NB3, rank2)
            tbits = (b1 * 524288) + (b2 * 256) + b3
            res[...] = lax.bitcast_convert_type(
                jnp.full((16,), tbits, jnp.int32), jnp.float32)
            pltpu.sync_copy(res, out_hbm)

    return final_kernel(inbin, h2s, bracket)


def _select_threshold(acts, k):
    cands, hists = _sc_compact(acts, k)
    bracket = _sc_bracket(hists, k)
    inbin, h2s = _sc_filter(cands, bracket)
    tvec = _sc_final(inbin, h2s, bracket, k)
    return tvec[0:1]


def _encode_body(x_ref, w_ref, b_ref, out_ref):
    acc = jnp.dot(x_ref[...], w_ref[...],
                  preferred_element_type=jnp.float32,
                  precision=jax.lax.Precision.DEFAULT)
    out_ref[...] = jnp.maximum(acc + b_ref[...], 0.0)


def _encode(x, w_enc, b_enc, bn=1024):
    B, d_src = x.shape
    D = w_enc.shape[1]
    grid = (D // bn,)
    return pl.pallas_call(
        _encode_body,
        grid=grid,
        in_specs=[
            pl.BlockSpec((B, d_src), lambda j: (0, 0)),
            pl.BlockSpec((d_src, bn), lambda j: (0, j)),
            pl.BlockSpec((1, bn), lambda j: (0, j)),
        ],
        out_specs=pl.BlockSpec((B, bn), lambda j: (0, j)),
        out_shape=jax.ShapeDtypeStruct((B, D), jnp.float32),
    )(x, w_enc, b_enc.reshape(1, D))


def _decode_body(t_ref, a_ref, w_ref, bd_ref, out_ref):
    kstep = pl.program_id(1)
    t = t_ref[0]
    a = a_ref[...]
    a = jnp.where(a >= t, a, 0.0)
    contrib = jnp.dot(a, w_ref[...],
                      preferred_element_type=jnp.float32,
                      precision=jax.lax.Precision.DEFAULT)

    @pl.when(kstep == 0)
    def _():
        out_ref[...] = contrib + bd_ref[...]

    @pl.when(kstep > 0)
    def _():
        out_ref[...] += contrib


def _decode(threshold, acts, w_dec, b_dec, bm=2048, bk=512):
    B, D = acts.shape
    d_tgt = w_dec.shape[1]
    bm = min(bm, B)
    grid = (B // bm, D // bk)
    return pl.pallas_call(
        _decode_body,
        grid=grid,
        in_specs=[
            pl.BlockSpec(memory_space=pltpu.SMEM),
            pl.BlockSpec((bm, bk), lambda i, j: (i, j)),
            pl.BlockSpec((bk, d_tgt), lambda i, j: (j, 0)),
            pl.BlockSpec((1, d_tgt), lambda i, j: (0, 0)),
        ],
        out_specs=pl.BlockSpec((bm, d_tgt), lambda i, j: (i, 0)),
        out_shape=jax.ShapeDtypeStruct((B, d_tgt), jnp.float32),
    )(threshold, acts, w_dec, b_dec.reshape(1, d_tgt))


def kernel(x_source, x_target, W_enc, b_enc, W_dec, b_dec):
    B = x_source.shape[0]
    D = W_enc.shape[1]
    bn = min(1024, D)
    acts = _encode(x_source, W_enc, b_enc, bn=bn)
    k = TOP_K_PER_ROW * B
    threshold = _select_threshold(acts, k)
    return _decode(threshold, acts, W_dec, b_dec, bk=min(512, D))


# speedup vs baseline: 1.0282x; 1.0003x over previous
import functools

import jax
import jax.numpy as jnp
from jax import lax
from jax.experimental import pallas as pl
from jax.experimental.pallas import tpu as pltpu
from jax.experimental.pallas import tpu_sc as plsc

TOP_K_PER_ROW = 64

NW = 32
NB1 = 4096
NB2 = 2048
NB3 = 256
SUBCAP = 2048
CAP = 16 * SUBCAP
SUBCAP2 = 256
CAP2 = 16 * SUBCAP2
SAMPLE_WIN_STRIDE = 8
SAMPLE_VREG_STRIDE = 4
MARGIN = 3

_mesh = functools.partial(
    plsc.VectorSubcoreMesh, core_axis_name="c", subcore_axis_name="s",
    num_cores=2)

_sc_params = pltpu.CompilerParams(needs_layout_passes=False)


def _wid():
    return lax.axis_index("s") * 2 + lax.axis_index("c")


def _zero_i32(ref, n):
    def body(i, _):
        ref[pl.ds(i * 16, 16)] = jnp.zeros((16,), jnp.int32)
        return 0
    lax.fori_loop(0, n // 16, body, 0)


def _zero_f32(ref, n):
    def body(i, _):
        ref[pl.ds(i * 16, 16)] = jnp.zeros((16,), jnp.float32)
        return 0
    lax.fori_loop(0, n // 16, body, 0)


def _extract(vec, lane):
    lanes = lax.iota(jnp.int32, 16)
    return jnp.sum(jnp.where(lanes == lane, vec, 0))


def _rank_search(href, nb, rank):
    nv = nb // 16
    lanes = lax.iota(jnp.int32, 16)

    def body(j, carry):
        found_bin, found_gt, cum_above = carry
        i = nv - 1 - j
        v = href[pl.ds(i * 16, 16)]
        incl = jnp.cumsum(v)
        s = jnp.sum(v)
        gt = cum_above + (s - incl)
        m = (gt < rank) & (gt + v >= rank)
        mi = jnp.where(m, 1, 0)
        hit = jnp.sum(mi)
        lane = jnp.sum(jnp.where(m, lanes, 0))
        gt_at = jnp.sum(jnp.where(m, gt, 0))
        found_bin = jnp.where(hit > 0, i * 16 + lane, found_bin)
        found_gt = jnp.where(hit > 0, gt_at, found_gt)
        return found_bin, found_gt, cum_above + s

    out = lax.fori_loop(0, nv, body,
                        (jnp.int32(0), jnp.int32(0), jnp.int32(0)))
    return out[0], out[1]


def _sc_compact(acts, k):
    B, D = acts.shape
    n = B * D
    rows = B // NW
    WR, WC = 8, 2048
    nwg_r = rows // WR
    nwg_c = D // WC
    nwin = nwg_r * nwg_c
    assert nwin % 2 == 0 or nwin == 1
    wsz = WR * WC
    samp_wins = list(range(0, nwin, SAMPLE_WIN_STRIDE))
    samp_elems = len(samp_wins) * (wsz // (16 * SAMPLE_VREG_STRIDE)) * 16
    r_samp = max(1, -(-(MARGIN * k * samp_elems) // n))

    @functools.partial(
        pl.kernel, mesh=_mesh(),
        compiler_params=pltpu.CompilerParams(needs_layout_passes=False,
                                             use_tc_tiling_on_sc=True),
        out_type=(jax.ShapeDtypeStruct((NW, CAP), jnp.float32),
                  jax.ShapeDtypeStruct((NW, NB1), jnp.int32)),
        scratch_types=[pltpu.VMEM((WR, WC), jnp.float32),
                       pltpu.VMEM((WR, WC), jnp.float32),
                       pltpu.VMEM((CAP,), jnp.float32),
                       pltpu.VMEM((NB1,), jnp.int32),
                       pltpu.SemaphoreType.DMA,
                       pltpu.SemaphoreType.DMA],
    )
    def compact_kernel(acts_hbm, cands_hbm, hist_hbm, win0, win1, vals,
                       hist, sem0, sem1):
        w = _wid()
        row_base = w * rows
        ones = jnp.ones((16,), jnp.int32)
        lanes = lax.iota(jnp.int32, 16)

        def src(g):
            gr = g // nwg_c
            gc = g % nwg_c
            return acts_hbm.at[pl.ds(row_base + gr * WR, WR),
                               pl.ds(gc * WC, WC)]

        _zero_i32(hist, NB1)

        def samp_win(gs, _):
            pltpu.sync_copy(src(gs * SAMPLE_WIN_STRIDE), win0)

            def samp_row(r, _):
                def samp_body(i, _):
                    for u in range(4):
                        c = (i * 4 + u) * SAMPLE_VREG_STRIDE * 16
                        v = win0[r, pl.ds(c, 16)]
                        bits = lax.bitcast_convert_type(v, jnp.int32)
                        b = jnp.right_shift(bits, 19)
                        plsc.addupdate_scatter(hist, [b], ones,
                                               mask=v > 0.0)
                    return 0
                lax.fori_loop(0, WC // (16 * SAMPLE_VREG_STRIDE * 4),
                              samp_body, 0)
                return 0
            lax.fori_loop(0, WR, samp_row, 0)
            return 0
        lax.fori_loop(0, len(samp_wins), samp_win, 0)
        bf, _gt = _rank_search(hist, NB1, jnp.int32(r_samp))
        floor_f = lax.bitcast_convert_type(
            jnp.full((16,), bf * 524288, jnp.int32), jnp.float32)

        _zero_f32(vals, CAP)
        lane_base = lanes * SUBCAP

        def start(g, buf, sem):
            pltpu.async_copy(src(g), buf, sem)

        def wait(g, buf, sem):
            pltpu.make_async_copy(src(g), buf, sem).wait()

        def process(buf, cnt_vec):
            unroll = 8

            def row_step(r, cnt):
                def vec_body(i, cnt):
                    vs, ms, mis = [], [], []
                    for u in range(unroll):
                        v = buf[r, pl.ds(i * (16 * unroll) + u * 16, 16)]
                        m = v >= floor_f
                        vs.append(v)
                        ms.append(m)
                        mis.append(jnp.where(m, 1, 0))
                    base = lane_base + jnp.minimum(cnt,
                                                   SUBCAP - 1 - unroll)
                    pre = base
                    for u in range(unroll):
                        plsc.store_scatter(vals, [pre], vs[u], mask=ms[u])
                        pre = pre + mis[u]
                    return cnt + (pre - base)
                return lax.fori_loop(0, WC // (16 * unroll), vec_body, cnt)
            return lax.fori_loop(0, WR, row_step, cnt_vec)

        start(0, win0, sem0)
        if nwin > 1:
            start(1, win1, sem1)

        def pair_body(h, cnt_vec):
            g0 = h * 2
            wait(g0, win0, sem0)
            cnt_vec = process(win0, cnt_vec)

            @pl.when(g0 + 2 < nwin)
            def _():
                start(g0 + 2, win0, sem0)
            wait(g0 + 1, win1, sem1)
            cnt_vec = process(win1, cnt_vec)

            @pl.when(g0 + 3 < nwin)
            def _():
                start(g0 + 3, win1, sem1)
            return cnt_vec

        cnt0 = jnp.zeros((16,), jnp.int32)
        if nwin == 1:
            wait(0, win0, sem0)
            process(win0, cnt0)
        else:
            lax.fori_loop(0, nwin // 2, pair_body, cnt0)

        _zero_i32(hist, NB1)

        def chist_body(i, _):
            for u in range(4):
                v = vals[pl.ds((i * 4 + u) * 16, 16)]
                bits = lax.bitcast_convert_type(v, jnp.int32)
                b = jnp.right_shift(bits, 19)
                plsc.addupdate_scatter(hist, [b], ones, mask=v >= floor_f)
            return 0
        lax.fori_loop(0, CAP // 64, chist_body, 0)

        pltpu.sync_copy(vals, cands_hbm.at[w])
        pltpu.sync_copy(hist, hist_hbm.at[w])

    return compact_kernel(acts)


def _rows_pipelined(hbm, buf0, buf1, sem0, sem1, nrows, process):
    def start(r, buf, sem):
        pltpu.async_copy(hbm.at[r], buf, sem)

    def wait(r, buf, sem):
        pltpu.make_async_copy(hbm.at[r], buf, sem).wait()

    start(0, buf0, sem0)
    if nrows > 1:
        start(1, buf1, sem1)

    def pair(h, _):
        r0 = h * 2
        wait(r0, buf0, sem0)
        process(buf0)

        @pl.when(r0 + 2 < nrows)
        def _():
            start(r0 + 2, buf0, sem0)
        wait(r0 + 1, buf1, sem1)
        process(buf1)

        @pl.when(r0 + 3 < nrows)
        def _():
            start(r0 + 3, buf1, sem1)
        return 0
    lax.fori_loop(0, nrows // 2, pair, 0)


def _add_rows(acc, row, nb):
    def body(i, _):
        for u in range(4):
            sl = pl.ds((i * 4 + u) * 16, 16)
            acc[sl] = acc[sl] + row[sl]
        return 0
    lax.fori_loop(0, nb // 64, body, 0)


def _sc_bracket(hists, k):
    @functools.partial(
        pl.kernel, mesh=_mesh(), compiler_params=_sc_params,
        out_type=jax.ShapeDtypeStruct((16,), jnp.int32),
        scratch_types=[pltpu.VMEM((NB1,), jnp.int32),
                       pltpu.VMEM((NB1,), jnp.int32),
                       pltpu.VMEM((NB1,), jnp.int32),
                       pltpu.VMEM((16,), jnp.int32),
                       pltpu.SemaphoreType.DMA,
                       pltpu.SemaphoreType.DMA],
    )
    def bracket_kernel(hists_hbm, out_hbm, acc, row0, row1, res, sem0,
                       sem1):
        w = _wid()

        @pl.when(w == 0)
        def _():
            _zero_i32(acc, NB1)
            _rows_pipelined(hists_hbm, row0, row1, sem0, sem1, NW,
                            lambda row: _add_rows(acc, row, NB1))

            b1, gt1 = _rank_search(acc, NB1, jnp.int32(k))
            lanes = lax.iota(jnp.int32, 16)
            res[...] = jnp.where(lanes == 0, b1,
                                 jnp.where(lanes == 1, gt1, 0))
            pltpu.sync_copy(res, out_hbm)

    return bracket_kernel(hists)


def _sc_filter(cands, bracket):
    @functools.partial(
        pl.kernel, mesh=_mesh(), compiler_params=_sc_params,
        out_type=(jax.ShapeDtypeStruct((NW, CAP2), jnp.float32),
                  jax.ShapeDtypeStruct((NW, NB2), jnp.int32)),
        scratch_types=[pltpu.VMEM((CAP,), jnp.float32),
                       pltpu.VMEM((CAP2,), jnp.float32),
                       pltpu.VMEM((NB2,), jnp.int32),
                       pltpu.VMEM((16,), jnp.int32)],
    )
    def filter_kernel(cands_hbm, brk_hbm, inbin_hbm, h2_hbm, cbuf, inbin,
                      h2, binfo):
        w = _wid()
        pltpu.sync_copy(cands_hbm.at[w], cbuf)
        pltpu.sync_copy(brk_hbm, binfo)
        b1 = _extract(binfo[...], 0)
        b1v = jnp.full((16,), b1, jnp.int32)
        lanes = lax.iota(jnp.int32, 16)
        lane_base = lanes * SUBCAP2
        ones = jnp.ones((16,), jnp.int32)

        _zero_f32(inbin, CAP2)

        def body(i, cnt):
            for u in range(4):
                v = cbuf[pl.ds((i * 4 + u) * 16, 16)]
                bits = lax.bitcast_convert_type(v, jnp.int32)
                m = jnp.right_shift(bits, 19) == b1v
                mi = jnp.where(m, 1, 0)
                idx = lane_base + jnp.minimum(cnt, SUBCAP2 - 1)
                plsc.store_scatter(inbin, [idx], v, mask=m)
                cnt = cnt + mi
            return cnt
        lax.fori_loop(0, CAP // 64, body, jnp.zeros((16,), jnp.int32))

        _zero_i32(h2, NB2)

        def h2_body(i, _):
            v = inbin[pl.ds(i * 16, 16)]
            bits = lax.bitcast_convert_type(v, jnp.int32)
            m = jnp.right_shift(bits, 19) == b1v
            b2 = jnp.right_shift(bits, 8) & 0x7FF
            plsc.addupdate_scatter(h2, [b2], ones, mask=m)
            return 0
        lax.fori_loop(0, CAP2 // 16, h2_body, 0)

        pltpu.sync_copy(inbin, inbin_hbm.at[w])
        pltpu.sync_copy(h2, h2_hbm.at[w])

    return filter_kernel(cands, bracket)


def _sc_final(inbin, h2s, bracket, k):
    @functools.partial(
        pl.kernel, mesh=_mesh(), compiler_params=_sc_params,
        out_type=jax.ShapeDtypeStruct((16,), jnp.float32),
        scratch_types=[pltpu.VMEM((NB2,), jnp.int32),
                       pltpu.VMEM((NB2,), jnp.int32),
                       pltpu.VMEM((NB2,), jnp.int32),
                       pltpu.VMEM((CAP2,), jnp.float32),
                       pltpu.VMEM((CAP2,), jnp.float32),
                       pltpu.VMEM((NB3,), jnp.int32),
                       pltpu.VMEM((16,), jnp.int32),
                       pltpu.VMEM((16,), jnp.float32),
                       pltpu.SemaphoreType.DMA,
                       pltpu.SemaphoreType.DMA],
    )
    def final_kernel(inbin_hbm, h2s_hbm, brk_hbm, out_hbm, acc, row0, row1,
                     cbuf0, cbuf1, h3, binfo, res, sem0, sem1):
        w = _wid()

        @pl.when(w == 0)
        def _():
            pltpu.sync_copy(brk_hbm, binfo)
            b1 = _extract(binfo[...], 0)
            gt1 = _extract(binfo[...], 1)
            rank1 = jnp.int32(k) - gt1
            b1v = jnp.full((16,), b1, jnp.int32)
            ones = jnp.ones((16,), jnp.int32)

            _zero_i32(acc, NB2)
            _rows_pipelined(h2s_hbm, row0, row1, sem0, sem1, NW,
                            lambda row: _add_rows(acc, row, NB2))

            b2, gt2 = _rank_search(acc, NB2, rank1)
            rank2 = rank1 - gt2
            b2v = jnp.full((16,), b2, jnp.int32)

            _zero_i32(h3, NB3)

            def h3_rows(cbuf):
                def h3_body(i, _):
                    for u in range(4):
                        v = cbuf[pl.ds((i * 4 + u) * 16, 16)]
                        bits = lax.bitcast_convert_type(v, jnp.int32)
                        m = ((jnp.right_shift(bits, 19) == b1v)
                             & ((jnp.right_shift(bits, 8) & 0x7FF) == b2v))
                        b3 = bits & 0xFF
                        plsc.addupdate_scatter(h3, [b3], ones, mask=m)
                    return 0
                lax.fori_loop(0, CAP2 // 64, h3_body, 0)
            _rows_pipelined(inbin_hbm, cbuf0, cbuf1, sem0, sem1, NW,
                            h3_rows)

            b3, _gt3 = _rank_search(h3, NB3, rank2)
            tbits = (b1 * 524288) + (b2 * 256) + b3
            res[...] = lax.bitcast_convert_type(
                jnp.full((16,), tbits, jnp.int32), jnp.float32)
            pltpu.sync_copy(res, out_hbm)

    return final_kernel(inbin, h2s, bracket)


def _select_threshold(acts, k):
    cands, hists = _sc_compact(acts, k)
    bracket = _sc_bracket(hists, k)
    inbin, h2s = _sc_filter(cands, bracket)
    tvec = _sc_final(inbin, h2s, bracket, k)
    return tvec[0:1]


def _encode_body(x_ref, w_ref, b_ref, out_ref):
    acc = jnp.dot(x_ref[...], w_ref[...],
                  preferred_element_type=jnp.float32,
                  precision=jax.lax.Precision.DEFAULT)
    out_ref[...] = jnp.maximum(acc + b_ref[...], 0.0)


def _encode(x, w_enc, b_enc, bn=1024):
    B, d_src = x.shape
    D = w_enc.shape[1]
    grid = (D // bn,)
    return pl.pallas_call(
        _encode_body,
        grid=grid,
        in_specs=[
            pl.BlockSpec((B, d_src), lambda j: (0, 0)),
            pl.BlockSpec((d_src, bn), lambda j: (0, j)),
            pl.BlockSpec((1, bn), lambda j: (0, j)),
        ],
        out_specs=pl.BlockSpec((B, bn), lambda j: (0, j)),
        out_shape=jax.ShapeDtypeStruct((B, D), jnp.float32),
    )(x, w_enc, b_enc.reshape(1, D))


def _decode_body(t_ref, a_ref, w_ref, bd_ref, out_ref):
    kstep = pl.program_id(1)
    t = t_ref[0]
    a = a_ref[...]
    a = jnp.where(a >= t, a, 0.0)
    contrib = jnp.dot(a, w_ref[...],
                      preferred_element_type=jnp.float32,
                      precision=jax.lax.Precision.DEFAULT)

    @pl.when(kstep == 0)
    def _():
        out_ref[...] = contrib + bd_ref[...]

    @pl.when(kstep > 0)
    def _():
        out_ref[...] += contrib


def _decode(threshold, acts, w_dec, b_dec, bm=2048, bk=512):
    B, D = acts.shape
    d_tgt = w_dec.shape[1]
    bm = min(bm, B)
    grid = (B // bm, D // bk)
    return pl.pallas_call(
        _decode_body,
        grid=grid,
        in_specs=[
            pl.BlockSpec(memory_space=pltpu.SMEM),
            pl.BlockSpec((bm, bk), lambda i, j: (i, j)),
            pl.BlockSpec((bk, d_tgt), lambda i, j: (j, 0)),
            pl.BlockSpec((1, d_tgt), lambda i, j: (0, 0)),
        ],
        out_specs=pl.BlockSpec((bm, d_tgt), lambda i, j: (i, 0)),
        out_shape=jax.ShapeDtypeStruct((B, d_tgt), jnp.float32),
    )(threshold, acts, w_dec, b_dec.reshape(1, d_tgt))


def kernel(x_source, x_target, W_enc, b_enc, W_dec, b_dec):
    B = x_source.shape[0]
    D = W_enc.shape[1]
    bn = min(1024, D)
    acts = _encode(x_source, W_enc, b_enc, bn=bn)
    k = TOP_K_PER_ROW * B
    threshold = _select_threshold(acts, k)
    return _decode(threshold, acts, W_dec, b_dec, bk=min(512, D))
